# Initial kernel scaffold; baseline (speedup 1.0000x reference)
import jax
import jax.numpy as jnp
from jax.experimental import pallas as pl


def kernel(x, laplacian, bn1_gamma, bn1_beta, W1, b1, bn2_gamma, bn2_beta, W2, b2):
    raise NotImplementedError("write your pallas kernel here")


# speedup vs baseline: 1.3769x; 1.3769x over previous
import functools

import jax
import jax.numpy as jnp
from jax.experimental import pallas as pl
from jax.experimental.pallas import tpu as pltpu

N = 4096
F = 128
RB = 256
NB = N // RB


def _body(x_ref, l_ref, g1_ref, bt1_ref, w1_ref, b1_ref, g2_ref, bt2_ref,
          w2_ref, b2_ref, out_ref,
          lb, xbn, xbn_bf, x1a, x1a_bf, yb, y_bf, y1, y1_bf):
    p = pl.program_id(0)
    i = pl.program_id(1)
    rows = pl.ds(i * RB, RB)

    def bn_affine(v, g_ref, bt_ref):
        mean = jnp.mean(v, axis=0, keepdims=True)
        var = jnp.mean(jnp.square(v), axis=0, keepdims=True) - jnp.square(mean)
        scale = g_ref[...] * jax.lax.rsqrt(var + 1e-5)
        shift = bt_ref[...] - mean * scale
        return v * scale + shift

    def mm(a_bf16, b_bf16):
        return jnp.dot(a_bf16, b_bf16, preferred_element_type=jnp.float32)

    @pl.when(p == 0)
    def _pass0():
        @pl.when(i == 0)
        def _():
            xbnv = bn_affine(x_ref[...], g1_ref, bt1_ref)
            xbn[...] = xbnv
            xbn_bf[...] = xbnv.astype(jnp.bfloat16)

        lblk = l_ref[...].astype(jnp.bfloat16)
        lb[rows, :] = lblk
        x1a[rows, :] = mm(lblk, xbn_bf[...])

    @pl.when(p == 1)
    def _pass1():
        @pl.when(i == 0)
        def _():
            x1a_bf[...] = x1a[...].astype(jnp.bfloat16)

        x2 = 2.0 * mm(lb[rows, :], x1a_bf[...]) - xbn[rows, :]
        h = (jnp.dot(xbn[rows, :], w1_ref[0], preferred_element_type=jnp.float32)
             + jnp.dot(x1a[rows, :], w1_ref[1], preferred_element_type=jnp.float32)
             + jnp.dot(x2, w1_ref[2], preferred_element_type=jnp.float32)
             + b1_ref[...])
        yb[rows, :] = jnp.maximum(h, 0.0)

    @pl.when(p == 2)
    def _pass2():
        @pl.when(i == 0)
        def _():
            yv = bn_affine(yb[...], g2_ref, bt2_ref)
            yb[...] = yv
            y_bf[...] = yv.astype(jnp.bfloat16)

        y1[rows, :] = mm(lb[rows, :], y_bf[...])

    @pl.when(p == 3)
    def _pass3():
        @pl.when(i == 0)
        def _():
            y1_bf[...] = y1[...].astype(jnp.bfloat16)

        y2 = 2.0 * mm(lb[rows, :], y1_bf[...]) - yb[rows, :]
        out2 = (jnp.dot(yb[rows, :], w2_ref[0], preferred_element_type=jnp.float32)
                + jnp.dot(y1[rows, :], w2_ref[1], preferred_element_type=jnp.float32)
                + jnp.dot(y2, w2_ref[2], preferred_element_type=jnp.float32)
                + b2_ref[...])
        out_ref[rows, :] = jnp.maximum(xbn[rows, :] + out2, 0.0)


@functools.partial(jax.jit, static_argnames=("interpret",))
def _run(x, laplacian, g1, bt1, W1, b1, g2, bt2, W2, b2, interpret=False):
    full = pl.BlockSpec((N, F), lambda p, i: (0, 0))
    vec = pl.BlockSpec((1, F), lambda p, i: (0, 0))
    wspec = pl.BlockSpec(W1.shape, lambda p, i: (0, 0, 0))
    lspec = pl.BlockSpec((RB, N), lambda p, i: (jnp.where(p == 0, i, 0), 0))

    return pl.pallas_call(
        _body,
        grid=(4, NB),
        in_specs=[full, lspec, vec, vec, wspec, vec, vec, vec, wspec, vec],
        out_specs=full,
        out_shape=jax.ShapeDtypeStruct((N, F), jnp.float32),
        scratch_shapes=[
            pltpu.VMEM((N, N), jnp.bfloat16),
            pltpu.VMEM((N, F), jnp.float32),
            pltpu.VMEM((N, F), jnp.bfloat16),
            pltpu.VMEM((N, F), jnp.float32),
            pltpu.VMEM((N, F), jnp.bfloat16),
            pltpu.VMEM((N, F), jnp.float32),
            pltpu.VMEM((N, F), jnp.bfloat16),
            pltpu.VMEM((N, F), jnp.float32),
            pltpu.VMEM((N, F), jnp.bfloat16),
        ],
        compiler_params=pltpu.CompilerParams(
            dimension_semantics=("arbitrary", "arbitrary"),
            vmem_limit_bytes=110 * 1024 * 1024,
        ),
        interpret=interpret,
    )(x, laplacian, g1, bt1, W1, b1, g2, bt2, W2, b2)


def kernel(x, laplacian, bn1_gamma, bn1_beta, W1, b1, bn2_gamma, bn2_beta,
           W2, b2):
    r = lambda v: v.reshape(1, F)
    return _run(x, laplacian, r(bn1_gamma), r(bn1_beta), W1, r(b1),
                r(bn2_gamma), r(bn2_beta), W2, r(b2))


# speedup vs baseline: 1.4231x; 1.0335x over previous
import functools

import jax
import jax.numpy as jnp
from jax.experimental import pallas as pl
from jax.experimental.pallas import tpu as pltpu

N = 4096
F = 128
RB = 256
NB = N // RB


def _body(x_ref, l_ref, g1_ref, bt1_ref, w1_ref, b1_ref, g2_ref, bt2_ref,
          w2_ref, b2_ref, out_ref,
          lb, xbn, xbn_bf, x1a_bf, lx1, y_bf, y1, y1_bf, base2):
    p = pl.program_id(0)
    i = pl.program_id(1)
    rows = pl.ds(i * RB, RB)

    def bn_affine(v, g_ref, bt_ref):
        mean = jnp.mean(v, axis=0, keepdims=True)
        var = jnp.mean(jnp.square(v), axis=0, keepdims=True) - jnp.square(mean)
        scale = g_ref[...] * jax.lax.rsqrt(var + 1e-5)
        shift = bt_ref[...] - mean * scale
        return v * scale + shift

    def mm(a, b):
        return jnp.dot(a, b, preferred_element_type=jnp.float32)

    bf = lambda v: v.astype(jnp.bfloat16)

    @pl.when(p == 0)
    def _pass0():
        @pl.when(i == 0)
        def _():
            xbnv = bn_affine(x_ref[...], g1_ref, bt1_ref)
            xbn[...] = xbnv
            xbn_bf[...] = bf(xbnv)

        lblk = bf(l_ref[...])
        lb[rows, :] = lblk
        x1a_bf[rows, :] = bf(mm(lblk, xbn_bf[...]))

    @pl.when(p == 1)
    def _pass1():
        lx1[rows, :] = mm(lb[rows, :], x1a_bf[...])

    @pl.when(p == 2)
    def _pass2():
        @pl.when(i == 0)
        def _():
            w0m2 = bf(w1_ref[0] - w1_ref[2])
            w2x2 = bf(2.0 * w1_ref[2])
            h = (mm(xbn_bf[...], w0m2)
                 + mm(x1a_bf[...], bf(w1_ref[1]))
                 + mm(bf(lx1[...]), w2x2)
                 + b1_ref[...])
            out1 = jnp.maximum(h, 0.0)
            y_bf[...] = bf(bn_affine(out1, g2_ref, bt2_ref))

        y1[rows, :] = mm(lb[rows, :], y_bf[...])

    @pl.when(p == 3)
    def _pass3():
        @pl.when(i == 0)
        def _():
            y1b = bf(y1[...])
            y1_bf[...] = y1b
            base2[...] = (mm(y_bf[...], bf(w2_ref[0] - w2_ref[2]))
                          + mm(y1b, bf(w2_ref[1]))
                          + b2_ref[...])

        t = mm(lb[rows, :], y1_bf[...])
        out2 = base2[rows, :] + mm(bf(t), bf(2.0 * w2_ref[2]))
        out_ref[rows, :] = jnp.maximum(xbn[rows, :] + out2, 0.0)


@functools.partial(jax.jit, static_argnames=("interpret",))
def _run(x, laplacian, g1, bt1, W1, b1, g2, bt2, W2, b2, interpret=False):
    full = pl.BlockSpec((N, F), lambda p, i: (0, 0))
    vec = pl.BlockSpec((1, F), lambda p, i: (0, 0))
    wspec = pl.BlockSpec(W1.shape, lambda p, i: (0, 0, 0))
    lspec = pl.BlockSpec((RB, N), lambda p, i: (jnp.where(p == 0, i, 0), 0))

    return pl.pallas_call(
        _body,
        grid=(4, NB),
        in_specs=[full, lspec, vec, vec, wspec, vec, vec, vec, wspec, vec],
        out_specs=full,
        out_shape=jax.ShapeDtypeStruct((N, F), jnp.float32),
        scratch_shapes=[
            pltpu.VMEM((N, N), jnp.bfloat16),
            pltpu.VMEM((N, F), jnp.float32),
            pltpu.VMEM((N, F), jnp.bfloat16),
            pltpu.VMEM((N, F), jnp.bfloat16),
            pltpu.VMEM((N, F), jnp.float32),
            pltpu.VMEM((N, F), jnp.bfloat16),
            pltpu.VMEM((N, F), jnp.float32),
            pltpu.VMEM((N, F), jnp.bfloat16),
            pltpu.VMEM((N, F), jnp.float32),
        ],
        compiler_params=pltpu.CompilerParams(
            dimension_semantics=("arbitrary", "arbitrary"),
            vmem_limit_bytes=110 * 1024 * 1024,
        ),
        interpret=interpret,
    )(x, laplacian, g1, bt1, W1, b1, g2, bt2, W2, b2)


def kernel(x, laplacian, bn1_gamma, bn1_beta, W1, b1, bn2_gamma, bn2_beta,
           W2, b2):
    r = lambda v: v.reshape(1, F)
    return _run(x, laplacian, r(bn1_gamma), r(bn1_beta), W1, r(b1),
                r(bn2_gamma), r(bn2_beta), W2, r(b2))
